# baseline (speedup 1.0000x reference)
import functools

import jax
import jax.numpy as jnp
from jax import lax
from jax.experimental import pallas as pl
from jax.experimental.pallas import tpu as pltpu
from jax.experimental.pallas import tpu_sc as plsc

B = 16384
D = 32
PK = 128 // D
NC = 2
NS = 16
NW = NC * NS
BPW = B // NW
RPW = BPW // PK


def _gather_body(u_tab, i_tab, u_idx, i_idx, ue_out, ie_out,
                 idx_u, idx_i, rows_u, rows_i, sem):
    wid = lax.axis_index("s") * NC + lax.axis_index("c")
    base = wid * BPW
    pltpu.sync_copy(u_idx.at[pl.ds(base, BPW)], idx_u)
    pltpu.sync_copy(i_idx.at[pl.ds(base, BPW)], idx_i)

    def body(c, carry):
        vec_u = idx_u[pl.ds(c * 16, 16)]
        vec_i = idx_i[pl.ds(c * 16, 16)]
        for k in range(16):
            row = c * (16 // PK) + k // PK
            lane = (k % PK) * D
            pltpu.async_copy(u_tab.at[vec_u[k]],
                             rows_u.at[row, pl.ds(lane, D)], sem)
            pltpu.async_copy(i_tab.at[vec_i[k]],
                             rows_i.at[row, pl.ds(lane, D)], sem)
        return carry

    lax.fori_loop(0, BPW // 16, body, 0)
    pltpu.make_async_copy(ue_out.at[pl.ds(0, RPW)], rows_u, sem).wait()
    pltpu.make_async_copy(ie_out.at[pl.ds(0, RPW)], rows_i, sem).wait()
    pltpu.sync_copy(rows_u, ue_out.at[pl.ds(wid * RPW, RPW)])
    pltpu.sync_copy(rows_i, ie_out.at[pl.ds(wid * RPW, RPW)])


_gather = functools.partial(
    pl.kernel,
    mesh=plsc.VectorSubcoreMesh(core_axis_name="c", subcore_axis_name="s"),
    out_type=(
        jax.ShapeDtypeStruct((B // PK, PK * D), jnp.float32),
        jax.ShapeDtypeStruct((B // PK, PK * D), jnp.float32),
    ),
    scratch_types=[
        pltpu.VMEM((BPW,), jnp.int32),
        pltpu.VMEM((BPW,), jnp.int32),
        pltpu.VMEM((RPW, PK * D), jnp.float32),
        pltpu.VMEM((RPW, PK * D), jnp.float32),
        pltpu.SemaphoreType.DMA,
    ],
)(_gather_body)


BLK = 2048


def _mlp_body(ue, ie, w1u, w1i, b1, w2, b2, w3, b3, out):
    h = jnp.dot(ue[...], w1u[...], preferred_element_type=jnp.float32)
    h = h + jnp.dot(ie[...], w1i[...], preferred_element_type=jnp.float32)
    h = jnp.maximum(h + b1[...], 0.0)
    h = jnp.dot(h, w2[...], preferred_element_type=jnp.float32)
    h = jnp.maximum(h + b2[...], 0.0)
    s = jnp.dot(h, w3[...], preferred_element_type=jnp.float32) + b3[...]
    out[...] = jax.nn.sigmoid(s)


_mlp = pl.pallas_call(
    _mlp_body,
    grid=(B // BLK,),
    in_specs=[
        pl.BlockSpec((BLK, D), lambda b: (b, 0)),
        pl.BlockSpec((BLK, D), lambda b: (b, 0)),
        pl.BlockSpec((D, 64), lambda b: (0, 0)),
        pl.BlockSpec((D, 64), lambda b: (0, 0)),
        pl.BlockSpec((1, 64), lambda b: (0, 0)),
        pl.BlockSpec((64, 32), lambda b: (0, 0)),
        pl.BlockSpec((1, 32), lambda b: (0, 0)),
        pl.BlockSpec((32, 1), lambda b: (0, 0)),
        pl.BlockSpec((1, 1), lambda b: (0, 0)),
    ],
    out_specs=pl.BlockSpec((BLK, 1), lambda b: (b, 0)),
    out_shape=jax.ShapeDtypeStruct((B, 1), jnp.float32),
)


def kernel(u, i, user_emb, item_emb, W1, b1, W2, b2, W3, b3):
    u32 = u.astype(jnp.int32)
    i32 = i.astype(jnp.int32)
    ue_p, ie_p = _gather(user_emb, item_emb, u32, i32)
    ue = ue_p.reshape(B, D)
    ie = ie_p.reshape(B, D)
    w1u = W1[:, :D].T
    w1i = W1[:, D:].T
    out = _mlp(ue, ie, w1u, w1i, b1.reshape(1, -1), W2.T, b2.reshape(1, -1),
               W3.T, b3.reshape(1, 1))
    return out.reshape(B)


# speedup vs baseline: 1.0056x; 1.0056x over previous
import functools

import jax
import jax.numpy as jnp
from jax import lax
from jax.experimental import pallas as pl
from jax.experimental.pallas import tpu as pltpu
from jax.experimental.pallas import tpu_sc as plsc

B = 16384
D = 32
NC = 2
NS = 16
NW = NC * NS
BPW = B // NW


def _gather_body(u_tab, i_tab, u_idx, i_idx, ue_out, ie_out,
                 idx_s, idx_v, rows_v, sem):
    wid = lax.axis_index("s") * NC + lax.axis_index("c")
    base = wid * BPW
    for tab, idx_hbm, out in ((u_tab, u_idx, ue_out), (i_tab, i_idx, ie_out)):
        pltpu.sync_copy(idx_hbm.at[pl.ds(base, BPW)], idx_v)

        def body(c, carry, tab=tab):
            vec = idx_v[pl.ds(c * 16, 16)]
            for k in range(16):
                pltpu.async_copy(tab.at[vec[k]], rows_v.at[c * 16 + k], sem)
            return carry

        lax.fori_loop(0, BPW // 16, body, 0)
        pltpu.make_async_copy(tab.at[pl.ds(0, BPW)], rows_v, sem).wait()
        pltpu.sync_copy(rows_v, out.at[pl.ds(base, BPW)])


_gather = functools.partial(
    pl.kernel,
    mesh=plsc.VectorSubcoreMesh(core_axis_name="c", subcore_axis_name="s"),
    out_type=(
        jax.ShapeDtypeStruct((B, D), jnp.float32),
        jax.ShapeDtypeStruct((B, D), jnp.float32),
    ),
    scratch_types=[
        pltpu.SMEM((BPW,), jnp.int32),
        pltpu.VMEM((BPW,), jnp.int32),
        pltpu.VMEM((BPW, D), jnp.float32),
        pltpu.SemaphoreType.DMA,
    ],
)(_gather_body)


BLK = 2048


def _mlp_body(ue, ie, w1u, w1i, b1, w2, b2, w3, b3, out):
    h = jnp.dot(ue[...], w1u[...], preferred_element_type=jnp.float32)
    h = h + jnp.dot(ie[...], w1i[...], preferred_element_type=jnp.float32)
    h = jnp.maximum(h + b1[...], 0.0)
    h = jnp.dot(h, w2[...], preferred_element_type=jnp.float32)
    h = jnp.maximum(h + b2[...], 0.0)
    s = jnp.dot(h, w3[...], preferred_element_type=jnp.float32) + b3[...]
    out[...] = jax.nn.sigmoid(s)


_mlp = pl.pallas_call(
    _mlp_body,
    grid=(B // BLK,),
    in_specs=[
        pl.BlockSpec((BLK, D), lambda b: (b, 0)),
        pl.BlockSpec((BLK, D), lambda b: (b, 0)),
        pl.BlockSpec((D, 64), lambda b: (0, 0)),
        pl.BlockSpec((D, 64), lambda b: (0, 0)),
        pl.BlockSpec((1, 64), lambda b: (0, 0)),
        pl.BlockSpec((64, 32), lambda b: (0, 0)),
        pl.BlockSpec((1, 32), lambda b: (0, 0)),
        pl.BlockSpec((32, 1), lambda b: (0, 0)),
        pl.BlockSpec((1, 1), lambda b: (0, 0)),
    ],
    out_specs=pl.BlockSpec((BLK, 1), lambda b: (b, 0)),
    out_shape=jax.ShapeDtypeStruct((B, 1), jnp.float32),
)


def kernel(u, i, user_emb, item_emb, W1, b1, W2, b2, W3, b3):
    u32 = u.astype(jnp.int32)
    i32 = i.astype(jnp.int32)
    ue, ie = _gather(user_emb, item_emb, u32, i32)
    w1u = W1[:, :D].T
    w1i = W1[:, D:].T
    out = _mlp(ue, ie, w1u, w1i, b1.reshape(1, -1), W2.T, b2.reshape(1, -1),
               W3.T, b3.reshape(1, 1))
    return out.reshape(B)
